# baseline (speedup 1.0000x reference)
import functools

import jax
import jax.numpy as jnp
from jax import lax
from jax.experimental import pallas as pl
from jax.experimental.pallas import tpu as pltpu
from jax.experimental.pallas import tpu_sc as plsc

N_SHIFT = 128
RG = 4
NBUF = 4


def kernel(x):
    B, R, T = x.shape
    rows = B * R
    body = T - N_SHIFT
    n_workers = 32
    rows_per_w = rows // n_workers
    n_chunks = rows_per_w // RG

    mesh = plsc.VectorSubcoreMesh(core_axis_name="c", subcore_axis_name="s")

    @functools.partial(
        pl.kernel,
        mesh=mesh,
        out_type=jax.ShapeDtypeStruct((rows, T), jnp.float32),
        scratch_types=(
            [pltpu.VMEM_SHARED((16, NBUF, RG, T), jnp.float32)]
            + [pltpu.SemaphoreType.DMA for _ in range(2 * NBUF)]
        ),
    )
    def sc_shift(x_hbm, out_hbm, shared, *sems):
        rsem = sems[:NBUF]
        wsem = sems[NBUF:]
        c = lax.axis_index("c")
        s = lax.axis_index("s")
        wid = s * 2 + c
        row0 = wid * rows_per_w
        read_h = {}
        write_h = {}

        def issue_read(ci):
            b = ci % NBUF
            r = pl.ds(row0 + ci * RG, RG)
            h1 = pltpu.async_copy(
                x_hbm.at[r, pl.ds(0, body)],
                shared.at[s, b, :, pl.ds(N_SHIFT, body)],
                rsem[b],
            )
            h2 = pltpu.async_copy(
                x_hbm.at[r, pl.ds(body, N_SHIFT)],
                shared.at[s, b, :, pl.ds(0, N_SHIFT)],
                rsem[b],
            )
            read_h[ci] = (h1, h2)

        def issue_write(ci):
            b = ci % NBUF
            r = pl.ds(row0 + ci * RG, RG)
            write_h[ci] = pltpu.async_copy(
                shared.at[s, b], out_hbm.at[r, :], wsem[b]
            )

        for ci in range(min(NBUF - 1, n_chunks)):
            issue_read(ci)
        for ci in range(n_chunks):
            for h in read_h.pop(ci):
                h.wait()
            issue_write(ci)
            nxt = ci + NBUF - 1
            if nxt < n_chunks:
                if ci >= 1:
                    write_h.pop(ci - 1).wait()
                issue_read(nxt)
        for ci in sorted(write_h):
            write_h.pop(ci).wait()

    out = sc_shift(x.reshape(rows, T))
    return out.reshape(B, R, T)


# speedup vs baseline: 1.0144x; 1.0144x over previous
import functools

import jax
import jax.numpy as jnp
from jax import lax
from jax.experimental import pallas as pl
from jax.experimental.pallas import tpu as pltpu
from jax.experimental.pallas import tpu_sc as plsc

N_SHIFT = 128
RG = 8
NBUF = 2


def kernel(x):
    B, R, T = x.shape
    rows = B * R
    body = T - N_SHIFT
    n_workers = 32
    rows_per_w = rows // n_workers
    n_chunks = rows_per_w // RG

    mesh = plsc.VectorSubcoreMesh(core_axis_name="c", subcore_axis_name="s")

    @functools.partial(
        pl.kernel,
        mesh=mesh,
        out_type=jax.ShapeDtypeStruct((rows, T), jnp.float32),
        scratch_types=(
            [pltpu.VMEM_SHARED((16, NBUF, RG, T), jnp.float32)]
            + [pltpu.SemaphoreType.DMA for _ in range(2 * NBUF)]
        ),
    )
    def sc_shift(x_hbm, out_hbm, shared, *sems):
        rsem = sems[:NBUF]
        wsem = sems[NBUF:]
        c = lax.axis_index("c")
        s = lax.axis_index("s")
        wid = c * 16 + s
        row0 = wid * rows_per_w
        read_h = {}
        write_h = {}

        def issue_read(ci):
            b = ci % NBUF
            r = pl.ds(row0 + ci * RG, RG)
            h1 = pltpu.async_copy(
                x_hbm.at[r, pl.ds(0, body)],
                shared.at[s, b, :, pl.ds(N_SHIFT, body)],
                rsem[b],
            )
            h2 = pltpu.async_copy(
                x_hbm.at[r, pl.ds(body, N_SHIFT)],
                shared.at[s, b, :, pl.ds(0, N_SHIFT)],
                rsem[b],
            )
            read_h[ci] = (h1, h2)

        def issue_write(ci):
            b = ci % NBUF
            r = pl.ds(row0 + ci * RG, RG)
            write_h[ci] = pltpu.async_copy(
                shared.at[s, b], out_hbm.at[r, :], wsem[b]
            )

        for ci in range(min(NBUF - 1, n_chunks)):
            issue_read(ci)
        for ci in range(n_chunks):
            for h in read_h.pop(ci):
                h.wait()
            issue_write(ci)
            nxt = ci + NBUF - 1
            if nxt < n_chunks:
                if ci >= 1:
                    write_h.pop(ci - 1).wait()
                issue_read(nxt)
        for ci in sorted(write_h):
            write_h.pop(ci).wait()

    out = sc_shift(x.reshape(rows, T))
    return out.reshape(B, R, T)
